# baseline (speedup 1.0000x reference)
import functools

import jax
import jax.numpy as jnp
from jax import lax
from jax.experimental import pallas as pl
from jax.experimental.pallas import tpu as pltpu
from jax.experimental.pallas import tpu_sc as plsc

N = 10000
E = 320000
H = 128
S = 16
EA = 16
HID = 128
MSG = 64

NC = 2
NS = 16
NW = NC * NS
EPW = E // NW
GC = 80
NCHUNK = EPW // GC

_MESH = plsc.VectorSubcoreMesh(core_axis_name="c", subcore_axis_name="s")
_SC_PARAMS = pltpu.CompilerParams(use_tc_tiling_on_sc=False)


MAIN = NCHUNK - (NCHUNK % 2)


def _gather_body(h_hbm, xs_hbm, src_hbm, dst_hbm,
                 hj_out, hi_out, xj_out, xi_out,
                 si0, si1, di0, di1, hj0, hj1, hi0, hi1, xj0, xj1, xi0, xi1,
                 semg0, semg1, semo0, semo1):
    c = lax.axis_index("c")
    s = lax.axis_index("s")
    wid = s * NC + c
    base = wid * EPW

    sib = (si0, si1)
    dib = (di0, di1)
    hjb = (hj0, hj1)
    hib = (hi0, hi1)
    xjb = (xj0, xj1)
    xib = (xi0, xi1)
    semg = (semg0, semg1)
    semo = (semo0, semo1)

    def load_idx(ci, b):
        off = base + ci * GC
        pltpu.sync_copy(src_hbm.at[pl.ds(off, GC)], sib[b])
        pltpu.sync_copy(dst_hbm.at[pl.ds(off, GC)], dib[b])

    def fire(b):
        pltpu.async_copy(h_hbm.at[sib[b]], hjb[b], semg[b])
        pltpu.async_copy(h_hbm.at[dib[b]], hib[b], semg[b])
        pltpu.async_copy(xs_hbm.at[sib[b]], xjb[b], semg[b])
        pltpu.async_copy(xs_hbm.at[dib[b]], xib[b], semg[b])

    def wait_gather(b):
        pltpu.make_async_copy(h_hbm.at[sib[b]], hjb[b], semg[b]).wait()
        pltpu.make_async_copy(h_hbm.at[dib[b]], hib[b], semg[b]).wait()
        pltpu.make_async_copy(xs_hbm.at[sib[b]], xjb[b], semg[b]).wait()
        pltpu.make_async_copy(xs_hbm.at[dib[b]], xib[b], semg[b]).wait()

    def fire_out(ci, b):
        off = base + ci * GC
        pltpu.async_copy(hjb[b], hj_out.at[pl.ds(off, GC)], semo[b])
        pltpu.async_copy(hib[b], hi_out.at[pl.ds(off, GC)], semo[b])
        pltpu.async_copy(xjb[b], xj_out.at[pl.ds(off, GC)], semo[b])
        pltpu.async_copy(xib[b], xi_out.at[pl.ds(off, GC)], semo[b])

    def wait_out(b):
        pltpu.make_async_copy(hjb[b], hj_out.at[pl.ds(base, GC)], semo[b]).wait()
        pltpu.make_async_copy(hib[b], hi_out.at[pl.ds(base, GC)], semo[b]).wait()
        pltpu.make_async_copy(xjb[b], xj_out.at[pl.ds(base, GC)], semo[b]).wait()
        pltpu.make_async_copy(xib[b], xi_out.at[pl.ds(base, GC)], semo[b]).wait()

    load_idx(0, 0)
    fire(0)

    def pair(j, carry):
        a = 2 * j
        load_idx(a + 1, 1)

        @pl.when(j > 0)
        def _drain1():
            wait_out(1)

        fire(1)
        wait_gather(0)
        fire_out(a, 0)

        @pl.when(a + 2 < MAIN)
        def _pre0():
            load_idx(a + 2, 0)

        wait_out(0)

        @pl.when(a + 2 < MAIN)
        def _fire0():
            fire(0)

        wait_gather(1)
        fire_out(a + 1, 1)
        return carry

    lax.fori_loop(0, MAIN // 2, pair, 0)
    wait_out(1)

    def tail(i, carry):
        load_idx(i, 0)
        fire(0)
        wait_gather(0)
        fire_out(i, 0)
        wait_out(0)
        return carry

    lax.fori_loop(MAIN, NCHUNK, tail, 0)


_sc_gather = pl.kernel(
    _gather_body,
    out_type=[
        jax.ShapeDtypeStruct((E, H), jnp.bfloat16),
        jax.ShapeDtypeStruct((E, H), jnp.bfloat16),
        jax.ShapeDtypeStruct((E, S), jnp.float32),
        jax.ShapeDtypeStruct((E, S), jnp.float32),
    ],
    mesh=_MESH,
    scratch_types=[
        pltpu.VMEM((GC,), jnp.int32),
        pltpu.VMEM((GC,), jnp.int32),
        pltpu.VMEM((GC,), jnp.int32),
        pltpu.VMEM((GC,), jnp.int32),
        pltpu.VMEM((GC, H), jnp.bfloat16),
        pltpu.VMEM((GC, H), jnp.bfloat16),
        pltpu.VMEM((GC, H), jnp.bfloat16),
        pltpu.VMEM((GC, H), jnp.bfloat16),
        pltpu.VMEM((GC, S), jnp.float32),
        pltpu.VMEM((GC, S), jnp.float32),
        pltpu.VMEM((GC, S), jnp.float32),
        pltpu.VMEM((GC, S), jnp.float32),
        pltpu.SemaphoreType.DMA,
        pltpu.SemaphoreType.DMA,
        pltpu.SemaphoreType.DMA,
        pltpu.SemaphoreType.DMA,
    ],
    compiler_params=_SC_PARAMS,
)


def _scatter_body(m_hbm, dst_hbm, zeros_hbm, out_hbm,
                  di0, di1, m0, m1, acc, seml0, seml1):
    c = lax.axis_index("c")
    s = lax.axis_index("s")
    wid = s * NC + c
    base = wid * EPW

    dib = (di0, di1)
    mb = (m0, m1)
    seml = (seml0, seml1)

    @pl.when(s == 0)
    def _init():
        pltpu.sync_copy(zeros_hbm, acc)

    plsc.subcore_barrier()

    def fire_load(ci, b):
        off = base + ci * GC
        pltpu.async_copy(dst_hbm.at[pl.ds(off, GC)], dib[b], seml[b])
        pltpu.async_copy(m_hbm.at[pl.ds(off, GC)], mb[b], seml[b])

    def wait_load(b):
        pltpu.make_async_copy(dst_hbm.at[pl.ds(base, GC)], dib[b], seml[b]).wait()
        pltpu.make_async_copy(m_hbm.at[pl.ds(base, GC)], mb[b], seml[b]).wait()

    def add(b):
        pltpu.sync_copy(mb[b], acc.at[dib[b]], add=True)

    fire_load(0, 0)

    def pair(j, carry):
        a = 2 * j
        fire_load(a + 1, 1)
        wait_load(0)
        add(0)

        @pl.when(a + 2 < MAIN)
        def _pre0():
            fire_load(a + 2, 0)

        wait_load(1)
        add(1)
        return carry

    lax.fori_loop(0, MAIN // 2, pair, 0)

    def tail(i, carry):
        fire_load(i, 0)
        wait_load(0)
        add(0)
        return carry

    lax.fori_loop(MAIN, NCHUNK, tail, 0)

    plsc.subcore_barrier()

    @pl.when(s == 0)
    def _emit():
        pltpu.sync_copy(acc, out_hbm.at[c])


_sc_scatter = pl.kernel(
    _scatter_body,
    out_type=jax.ShapeDtypeStruct((NC, N, MSG), jnp.float32),
    mesh=_MESH,
    scratch_types=[
        pltpu.VMEM((GC,), jnp.int32),
        pltpu.VMEM((GC,), jnp.int32),
        pltpu.VMEM((GC, MSG), jnp.float32),
        pltpu.VMEM((GC, MSG), jnp.float32),
        pltpu.VMEM_SHARED((N, MSG), jnp.float32),
        pltpu.SemaphoreType.DMA,
        pltpu.SemaphoreType.DMA,
    ],
    compiler_params=_SC_PARAMS,
)


BE = 3200


def _edge_mlp_body(hj, hi, xj, xi, ea,
                   W_es1, b_es1, W_es2, b_es2, W_bw, b_bw,
                   W_g1, b_g1, W_g2, b_g2, W_p1, b_p1, W_p2, b_p2,
                   m_out):
    f32 = jnp.float32
    dot = functools.partial(jnp.dot, preferred_element_type=f32)

    w1 = W_es1[...]
    z1 = (dot(ea[...], w1[0:EA, :]) + dot(xj[...], w1[EA:EA + S, :])
          + dot(xi[...], w1[EA + S:, :]) + b_es1[...])
    w_c = dot(W_es2[...], W_bw[...])
    c0 = dot(b_es2[...], W_bw[...]) + b_bw[...]
    t = dot(jax.nn.relu(z1), w_c) + c0
    b_e = jax.nn.softplus(t)

    bf16 = jnp.bfloat16
    wg = W_g1[...].astype(bf16)
    a = dot(hj[...], wg[0:H, :]) + dot(hi[...], wg[H:, :]) + b_g1[...]
    g_e = jax.nn.sigmoid(dot(jax.nn.relu(a), W_g2[...]) + b_g2[...])

    v = (dot(jax.nn.relu(dot(hj[...], W_p1[...].astype(bf16)) + b_p1[...]),
             W_p2[...]) + b_p2[...])
    m_out[...] = b_e * g_e * v


def _edge_mlp(hj, hi, xj, xi, ea, W_es1, b_es1, W_es2, b_es2, W_bw, b_bw,
              W_g1, b_g1, W_g2, b_g2, W_p1, b_p1, W_p2, b_p2):
    grid = (E // BE,)

    def eb(width):
        return pl.BlockSpec((BE, width), lambda i: (i, 0))

    def full(shape):
        return pl.BlockSpec(shape, lambda i: tuple(0 for _ in shape))

    return pl.pallas_call(
        _edge_mlp_body,
        grid=grid,
        in_specs=[
            eb(H), eb(H), eb(S), eb(S), eb(EA),
            full((EA + 2 * S, HID)), full((1, HID)),
            full((HID, HID)), full((1, HID)),
            full((HID, 1)), full((1, 1)),
            full((2 * H, HID)), full((1, HID)),
            full((HID, 1)), full((1, 1)),
            full((H, HID)), full((1, HID)),
            full((HID, MSG)), full((1, MSG)),
        ],
        out_specs=eb(MSG),
        out_shape=jax.ShapeDtypeStruct((E, MSG), jnp.float32),
    )(hj, hi, xj, xi, ea, W_es1, b_es1, W_es2, b_es2, W_bw, b_bw,
      W_g1, b_g1, W_g2, b_g2, W_p1, b_p1, W_p2, b_p2)


def _combine_body(p, out):
    out[...] = p[0] + p[1]


def _combine(partials):
    return pl.pallas_call(
        _combine_body,
        out_shape=jax.ShapeDtypeStruct((N, MSG), jnp.float32),
    )(partials)


def kernel(h, x_static, edge_attr_static, edge_index,
           W_es1, b_es1, W_es2, b_es2, W_bw, b_bw,
           W_g1, b_g1, W_g2, b_g2, W_p1, b_p1, W_p2, b_p2):
    src = edge_index[0]
    dst = edge_index[1]

    hj, hi, xj, xi = _sc_gather(h.astype(jnp.bfloat16), x_static, src, dst)

    m = _edge_mlp(
        hj, hi, xj, xi, edge_attr_static,
        W_es1, b_es1.reshape(1, HID), W_es2, b_es2.reshape(1, HID),
        W_bw, b_bw.reshape(1, 1),
        W_g1, b_g1.reshape(1, HID), W_g2, b_g2.reshape(1, 1),
        W_p1, b_p1.reshape(1, HID), W_p2, b_p2.reshape(1, MSG))

    zeros = jnp.zeros((N, MSG), jnp.float32)
    partials = _sc_scatter(m, dst, zeros)
    return _combine(partials)


# speedup vs baseline: 1.4852x; 1.4852x over previous
import functools

import jax
import jax.numpy as jnp
from jax import lax
from jax.experimental import pallas as pl
from jax.experimental.pallas import tpu as pltpu
from jax.experimental.pallas import tpu_sc as plsc

N = 10000
E = 320000
H = 128
S = 16
EA = 16
HID = 128
MSG = 64

NC = 2
NS = 16
NW = NC * NS
EPW = E // NW
GC = 80
NCHUNK = EPW // GC

_MESH = plsc.VectorSubcoreMesh(core_axis_name="c", subcore_axis_name="s")
_SC_PARAMS = pltpu.CompilerParams(use_tc_tiling_on_sc=False)


MAIN = NCHUNK - (NCHUNK % 2)


def _gather_body(h_hbm, xs_hbm, src_hbm, dst_hbm,
                 hj_out, hi_out, xj_out, xi_out,
                 si0, si1, di0, di1, hj0, hj1, hi0, hi1, xj0, xj1, xi0, xi1,
                 semg0, semg1, semo0, semo1):
    c = lax.axis_index("c")
    s = lax.axis_index("s")
    wid = s * NC + c
    base = wid * EPW

    sib = (si0, si1)
    dib = (di0, di1)
    hjb = (hj0, hj1)
    hib = (hi0, hi1)
    xjb = (xj0, xj1)
    xib = (xi0, xi1)
    semg = (semg0, semg1)
    semo = (semo0, semo1)

    def load_idx(ci, b):
        off = base + ci * GC
        pltpu.sync_copy(src_hbm.at[pl.ds(off, GC)], sib[b])
        pltpu.sync_copy(dst_hbm.at[pl.ds(off, GC)], dib[b])

    def fire(b):
        pltpu.async_copy(h_hbm.at[sib[b]], hjb[b], semg[b])
        pltpu.async_copy(h_hbm.at[dib[b]], hib[b], semg[b])
        pltpu.async_copy(xs_hbm.at[sib[b]], xjb[b], semg[b])
        pltpu.async_copy(xs_hbm.at[dib[b]], xib[b], semg[b])

    def wait_gather(b):
        pltpu.make_async_copy(h_hbm.at[sib[b]], hjb[b], semg[b]).wait()
        pltpu.make_async_copy(h_hbm.at[dib[b]], hib[b], semg[b]).wait()
        pltpu.make_async_copy(xs_hbm.at[sib[b]], xjb[b], semg[b]).wait()
        pltpu.make_async_copy(xs_hbm.at[dib[b]], xib[b], semg[b]).wait()

    def fire_out(ci, b):
        off = base + ci * GC
        pltpu.async_copy(hjb[b], hj_out.at[pl.ds(off, GC)], semo[b])
        pltpu.async_copy(hib[b], hi_out.at[pl.ds(off, GC)], semo[b])
        pltpu.async_copy(xjb[b], xj_out.at[pl.ds(off, GC)], semo[b])
        pltpu.async_copy(xib[b], xi_out.at[pl.ds(off, GC)], semo[b])

    def wait_out(b):
        pltpu.make_async_copy(hjb[b], hj_out.at[pl.ds(base, GC)], semo[b]).wait()
        pltpu.make_async_copy(hib[b], hi_out.at[pl.ds(base, GC)], semo[b]).wait()
        pltpu.make_async_copy(xjb[b], xj_out.at[pl.ds(base, GC)], semo[b]).wait()
        pltpu.make_async_copy(xib[b], xi_out.at[pl.ds(base, GC)], semo[b]).wait()

    load_idx(0, 0)
    fire(0)

    def pair(j, carry):
        a = 2 * j
        load_idx(a + 1, 1)

        @pl.when(j > 0)
        def _drain1():
            wait_out(1)

        fire(1)
        wait_gather(0)
        fire_out(a, 0)

        @pl.when(a + 2 < MAIN)
        def _pre0():
            load_idx(a + 2, 0)

        wait_out(0)

        @pl.when(a + 2 < MAIN)
        def _fire0():
            fire(0)

        wait_gather(1)
        fire_out(a + 1, 1)
        return carry

    lax.fori_loop(0, MAIN // 2, pair, 0)
    wait_out(1)

    def tail(i, carry):
        load_idx(i, 0)
        fire(0)
        wait_gather(0)
        fire_out(i, 0)
        wait_out(0)
        return carry

    lax.fori_loop(MAIN, NCHUNK, tail, 0)


_sc_gather = pl.kernel(
    _gather_body,
    out_type=[
        jax.ShapeDtypeStruct((E, H), jnp.float32),
        jax.ShapeDtypeStruct((E, H), jnp.float32),
        jax.ShapeDtypeStruct((E, S), jnp.float32),
        jax.ShapeDtypeStruct((E, S), jnp.float32),
    ],
    mesh=_MESH,
    scratch_types=[
        pltpu.VMEM((GC,), jnp.int32),
        pltpu.VMEM((GC,), jnp.int32),
        pltpu.VMEM((GC,), jnp.int32),
        pltpu.VMEM((GC,), jnp.int32),
        pltpu.VMEM((GC, H), jnp.float32),
        pltpu.VMEM((GC, H), jnp.float32),
        pltpu.VMEM((GC, H), jnp.float32),
        pltpu.VMEM((GC, H), jnp.float32),
        pltpu.VMEM((GC, S), jnp.float32),
        pltpu.VMEM((GC, S), jnp.float32),
        pltpu.VMEM((GC, S), jnp.float32),
        pltpu.VMEM((GC, S), jnp.float32),
        pltpu.SemaphoreType.DMA,
        pltpu.SemaphoreType.DMA,
        pltpu.SemaphoreType.DMA,
        pltpu.SemaphoreType.DMA,
    ],
    compiler_params=_SC_PARAMS,
)


def _scatter_body(m_hbm, dst_hbm, zeros_hbm, out_hbm,
                  di0, di1, m0, m1, acc, seml0, seml1):
    c = lax.axis_index("c")
    s = lax.axis_index("s")
    wid = s * NC + c
    base = wid * EPW

    dib = (di0, di1)
    mb = (m0, m1)
    seml = (seml0, seml1)

    @pl.when(s == 0)
    def _init():
        pltpu.sync_copy(zeros_hbm, acc)

    plsc.subcore_barrier()

    def fire_load(ci, b):
        off = base + ci * GC
        pltpu.async_copy(dst_hbm.at[pl.ds(off, GC)], dib[b], seml[b])
        pltpu.async_copy(m_hbm.at[pl.ds(off, GC)], mb[b], seml[b])

    def wait_load(b):
        pltpu.make_async_copy(dst_hbm.at[pl.ds(base, GC)], dib[b], seml[b]).wait()
        pltpu.make_async_copy(m_hbm.at[pl.ds(base, GC)], mb[b], seml[b]).wait()

    def add(b):
        pltpu.sync_copy(mb[b], acc.at[dib[b]], add=True)

    fire_load(0, 0)

    def pair(j, carry):
        a = 2 * j
        fire_load(a + 1, 1)
        wait_load(0)
        add(0)

        @pl.when(a + 2 < MAIN)
        def _pre0():
            fire_load(a + 2, 0)

        wait_load(1)
        add(1)
        return carry

    lax.fori_loop(0, MAIN // 2, pair, 0)

    def tail(i, carry):
        fire_load(i, 0)
        wait_load(0)
        add(0)
        return carry

    lax.fori_loop(MAIN, NCHUNK, tail, 0)

    plsc.subcore_barrier()

    @pl.when(s == 0)
    def _emit():
        pltpu.sync_copy(acc, out_hbm.at[c])


_sc_scatter = pl.kernel(
    _scatter_body,
    out_type=jax.ShapeDtypeStruct((NC, N, MSG), jnp.float32),
    mesh=_MESH,
    scratch_types=[
        pltpu.VMEM((GC,), jnp.int32),
        pltpu.VMEM((GC,), jnp.int32),
        pltpu.VMEM((GC, MSG), jnp.float32),
        pltpu.VMEM((GC, MSG), jnp.float32),
        pltpu.VMEM_SHARED((N, MSG), jnp.float32),
        pltpu.SemaphoreType.DMA,
        pltpu.SemaphoreType.DMA,
    ],
    compiler_params=_SC_PARAMS,
)


BE = 3200


def _edge_mlp_body(hj, hi, xj, xi, ea,
                   W_es1, b_es1, W_es2, b_es2, W_bw, b_bw,
                   W_g1, b_g1, W_g2, b_g2, W_p1, b_p1, W_p2, b_p2,
                   m_out):
    f32 = jnp.float32
    dot = functools.partial(jnp.dot, preferred_element_type=f32)

    w1 = W_es1[...]
    z1 = (dot(ea[...], w1[0:EA, :]) + dot(xj[...], w1[EA:EA + S, :])
          + dot(xi[...], w1[EA + S:, :]) + b_es1[...])
    w_c = dot(W_es2[...], W_bw[...])
    c0 = dot(b_es2[...], W_bw[...]) + b_bw[...]
    t = dot(jax.nn.relu(z1), w_c) + c0
    b_e = jax.nn.softplus(t)

    bf16 = jnp.bfloat16
    hj16 = hj[...].astype(bf16)
    hi16 = hi[...].astype(bf16)
    wg = W_g1[...]
    a = dot(hj16, wg[0:H, :]) + dot(hi16, wg[H:, :]) + b_g1[...]
    g_e = jax.nn.sigmoid(dot(jax.nn.relu(a), W_g2[...]) + b_g2[...])

    v = (dot(jax.nn.relu(dot(hj16, W_p1[...]) + b_p1[...]),
             W_p2[...]) + b_p2[...])
    m_out[...] = b_e * g_e * v


def _edge_mlp(hj, hi, xj, xi, ea, W_es1, b_es1, W_es2, b_es2, W_bw, b_bw,
              W_g1, b_g1, W_g2, b_g2, W_p1, b_p1, W_p2, b_p2):
    grid = (E // BE,)

    def eb(width):
        return pl.BlockSpec((BE, width), lambda i: (i, 0))

    def full(shape):
        return pl.BlockSpec(shape, lambda i: tuple(0 for _ in shape))

    return pl.pallas_call(
        _edge_mlp_body,
        grid=grid,
        in_specs=[
            eb(H), eb(H), eb(S), eb(S), eb(EA),
            full((EA + 2 * S, HID)), full((1, HID)),
            full((HID, HID)), full((1, HID)),
            full((HID, 1)), full((1, 1)),
            full((2 * H, HID)), full((1, HID)),
            full((HID, 1)), full((1, 1)),
            full((H, HID)), full((1, HID)),
            full((HID, MSG)), full((1, MSG)),
        ],
        out_specs=eb(MSG),
        out_shape=jax.ShapeDtypeStruct((E, MSG), jnp.float32),
    )(hj, hi, xj, xi, ea, W_es1, b_es1, W_es2, b_es2, W_bw, b_bw,
      W_g1, b_g1, W_g2, b_g2, W_p1, b_p1, W_p2, b_p2)


def _combine_body(p, out):
    out[...] = p[0] + p[1]


def _combine(partials):
    return pl.pallas_call(
        _combine_body,
        out_shape=jax.ShapeDtypeStruct((N, MSG), jnp.float32),
    )(partials)


def kernel(h, x_static, edge_attr_static, edge_index,
           W_es1, b_es1, W_es2, b_es2, W_bw, b_bw,
           W_g1, b_g1, W_g2, b_g2, W_p1, b_p1, W_p2, b_p2):
    src = edge_index[0]
    dst = edge_index[1]

    hj, hi, xj, xi = _sc_gather(h, x_static, src, dst)

    m = _edge_mlp(
        hj, hi, xj, xi, edge_attr_static,
        W_es1, b_es1.reshape(1, HID), W_es2, b_es2.reshape(1, HID),
        W_bw, b_bw.reshape(1, 1),
        W_g1.astype(jnp.bfloat16), b_g1.reshape(1, HID), W_g2,
        b_g2.reshape(1, 1),
        W_p1.astype(jnp.bfloat16), b_p1.reshape(1, HID), W_p2,
        b_p2.reshape(1, MSG))

    zeros = jnp.zeros((N, MSG), jnp.float32)
    partials = _sc_scatter(m, dst, zeros)
    return _combine(partials)
